# baseline (speedup 1.0000x reference)
import jax
import jax.numpy as jnp
from jax.experimental import pallas as pl
from jax.experimental.pallas import tpu as pltpu


def _fuzzy_chunk(x, parity, keep):
    blk, ho, w2 = x.shape
    w = w2 // 2
    wo = w // 2
    ws = x * (2.0 / 3.0)

    a = x[:, :, :w]
    b = x[:, :, w:]
    wa = ws[:, :, :w]
    wb = ws[:, :, w:]

    m1a = jnp.maximum(1.0 - jnp.abs(wa - 1.0), 0.0)
    m2a = jnp.maximum(1.0 - jnp.abs(wa - 2.0), 0.0)
    m1b = jnp.maximum(1.0 - jnp.abs(wb - 1.0), 0.0)
    m2b = jnp.maximum(1.0 - jnp.abs(wb - 2.0), 0.0)

    ds = (m1a - m2a) + (m1b - m2b)
    ds = ds + jnp.roll(ds, -1, axis=-1)
    u = jnp.where(ds >= 0.0, 1.0, 0.0)
    u = jnp.where(parity == 0, u, jnp.roll(u, 1, axis=-1))
    pick1 = u > 0.5

    da = jnp.where(pick1, m1a, m2a) * a
    db = jnp.where(pick1, m1b, m2b) * b
    den_f = da + db
    num_f = da * a + db * b
    den_f = den_f + jnp.roll(den_f, -1, axis=-1)
    num_f = num_f + jnp.roll(num_f, -1, axis=-1)

    stacked = jnp.concatenate(
        [num_f.reshape(blk * ho, w), den_f.reshape(blk * ho, w)], axis=0)
    r = jax.lax.dot(stacked, keep, precision=jax.lax.Precision.HIGHEST)
    num = r[: blk * ho].reshape(blk, ho, wo)
    den = r[blk * ho:].reshape(blk, ho, wo)

    return jnp.where(den == 0.0, 0.0, num / jnp.where(den == 0.0, 1.0, den))


def _fuzzy_kernel(x_ref, o_ref):
    blk, ho, w2 = x_ref.shape
    w = w2 // 2
    wo = w // 2
    parity = jax.lax.broadcasted_iota(jnp.int32, (1, 1, w), 2) % 2
    ki = jax.lax.broadcasted_iota(jnp.int32, (w, wo), 0)
    ji = jax.lax.broadcasted_iota(jnp.int32, (w, wo), 1)
    keep = (ki == 2 * ji).astype(jnp.float32)
    CB = 4
    for k in range(blk // CB):
        sl = pl.ds(k * CB, CB)
        o_ref[sl] = _fuzzy_chunk(x_ref[sl], parity, keep)


def kernel(x):
    B, C, H, W = x.shape
    Ho, Wo = H // 2, W // 2
    BLK = 64
    n = B * C
    xv = x.reshape(n, Ho, 2 * W)
    out = pl.pallas_call(
        _fuzzy_kernel,
        grid=(n // BLK,),
        in_specs=[pl.BlockSpec((BLK, Ho, 2 * W), lambda i: (i, 0, 0))],
        out_specs=pl.BlockSpec((BLK, Ho, Wo), lambda i: (i, 0, 0)),
        out_shape=jax.ShapeDtypeStruct((n, Ho, Wo), x.dtype),
        compiler_params=pltpu.CompilerParams(dimension_semantics=("parallel",)),
    )(xv)
    return out.reshape(B, C, Ho, Wo)


# speedup vs baseline: 1.0054x; 1.0054x over previous
import jax
import jax.numpy as jnp
from jax.experimental import pallas as pl
from jax.experimental.pallas import tpu as pltpu


def _fuzzy_chunk(x, parity, keep):
    blk, ho, w2 = x.shape
    w = w2 // 2
    wo = w // 2
    ws = x * (2.0 / 3.0)

    a = x[:, :, :w]
    b = x[:, :, w:]
    wa = ws[:, :, :w]
    wb = ws[:, :, w:]

    m1a = jnp.maximum(1.0 - jnp.abs(wa - 1.0), 0.0)
    m2a = jnp.maximum(1.0 - jnp.abs(wa - 2.0), 0.0)
    m1b = jnp.maximum(1.0 - jnp.abs(wb - 1.0), 0.0)
    m2b = jnp.maximum(1.0 - jnp.abs(wb - 2.0), 0.0)

    ds = (m1a - m2a) + (m1b - m2b)
    ds = ds + jnp.roll(ds, -1, axis=-1)
    ds = jnp.where(parity == 0, ds, jnp.roll(ds, 1, axis=-1))
    pick1 = ds >= 0.0

    da = jnp.where(pick1, m1a, m2a) * a
    db = jnp.where(pick1, m1b, m2b) * b
    den_f = da + db
    num_f = da * a + db * b
    den_f = den_f + jnp.roll(den_f, -1, axis=-1)
    num_f = num_f + jnp.roll(num_f, -1, axis=-1)

    stacked = jnp.concatenate(
        [num_f.reshape(blk * ho, w), den_f.reshape(blk * ho, w)], axis=0)
    r = jax.lax.dot(stacked, keep, precision=jax.lax.Precision.HIGHEST)
    num = r[: blk * ho].reshape(blk, ho, wo)
    den = r[blk * ho:].reshape(blk, ho, wo)

    return jnp.where(den == 0.0, 0.0, num / jnp.where(den == 0.0, 1.0, den))


def _fuzzy_kernel(x_ref, o_ref):
    blk, ho, w2 = x_ref.shape
    w = w2 // 2
    wo = w // 2
    parity = jax.lax.broadcasted_iota(jnp.int32, (1, 1, w), 2) % 2
    ki = jax.lax.broadcasted_iota(jnp.int32, (w, wo), 0)
    ji = jax.lax.broadcasted_iota(jnp.int32, (w, wo), 1)
    keep = (ki == 2 * ji).astype(jnp.float32)
    CB = 2
    for k in range(blk // CB):
        sl = pl.ds(k * CB, CB)
        o_ref[sl] = _fuzzy_chunk(x_ref[sl], parity, keep)


def kernel(x):
    B, C, H, W = x.shape
    Ho, Wo = H // 2, W // 2
    BLK = 64
    n = B * C
    xv = x.reshape(n, Ho, 2 * W)
    out = pl.pallas_call(
        _fuzzy_kernel,
        grid=(n // BLK,),
        in_specs=[pl.BlockSpec((BLK, Ho, 2 * W), lambda i: (i, 0, 0))],
        out_specs=pl.BlockSpec((BLK, Ho, Wo), lambda i: (i, 0, 0)),
        out_shape=jax.ShapeDtypeStruct((n, Ho, Wo), x.dtype),
        compiler_params=pltpu.CompilerParams(dimension_semantics=("parallel",)),
    )(xv)
    return out.reshape(B, C, Ho, Wo)


# speedup vs baseline: 1.0100x; 1.0045x over previous
import jax
import jax.numpy as jnp
from jax.experimental import pallas as pl
from jax.experimental.pallas import tpu as pltpu


def _fuzzy_chunk(x, parity, keep):
    blk, ho, w2 = x.shape
    w = w2 // 2
    wo = w // 2
    ws = x * (2.0 / 3.0)

    a = x[:, :, :w]
    b = x[:, :, w:]
    wa = ws[:, :, :w]
    wb = ws[:, :, w:]

    m1a = jnp.maximum(1.0 - jnp.abs(wa - 1.0), 0.0)
    m2a = jnp.maximum(1.0 - jnp.abs(wa - 2.0), 0.0)
    m1b = jnp.maximum(1.0 - jnp.abs(wb - 1.0), 0.0)
    m2b = jnp.maximum(1.0 - jnp.abs(wb - 2.0), 0.0)

    ds = (m1a - m2a) + (m1b - m2b)
    ds = ds + jnp.roll(ds, -1, axis=-1)
    ds = jnp.where(parity == 0, ds, jnp.roll(ds, 1, axis=-1))
    pick1 = ds >= 0.0

    da = jnp.where(pick1, m1a, m2a) * a
    db = jnp.where(pick1, m1b, m2b) * b
    den_f = da + db
    num_f = da * a + db * b
    den_f = den_f + jnp.roll(den_f, -1, axis=-1)
    num_f = num_f + jnp.roll(num_f, -1, axis=-1)

    stacked = jnp.concatenate(
        [num_f.reshape(blk * ho, w), den_f.reshape(blk * ho, w)], axis=0)
    r = jax.lax.dot(stacked, keep, precision=jax.lax.Precision.HIGHEST)
    num = r[: blk * ho].reshape(blk, ho, wo)
    den = r[blk * ho:].reshape(blk, ho, wo)

    return jnp.where(den == 0.0, 0.0, num / jnp.where(den == 0.0, 1.0, den))


def _fuzzy_kernel(x_ref, o_ref):
    blk, ho, w2 = x_ref.shape
    w = w2 // 2
    wo = w // 2
    parity = jax.lax.broadcasted_iota(jnp.int32, (1, 1, w), 2) % 2
    ki = jax.lax.broadcasted_iota(jnp.int32, (w, wo), 0)
    ji = jax.lax.broadcasted_iota(jnp.int32, (w, wo), 1)
    keep = (ki == 2 * ji).astype(jnp.float32)
    CB = 2
    for k in range(blk // CB):
        sl = pl.ds(k * CB, CB)
        o_ref[sl] = _fuzzy_chunk(x_ref[sl], parity, keep)


def kernel(x):
    B, C, H, W = x.shape
    Ho, Wo = H // 2, W // 2
    BLK = 128
    n = B * C
    xv = x.reshape(n, Ho, 2 * W)
    out = pl.pallas_call(
        _fuzzy_kernel,
        grid=(n // BLK,),
        in_specs=[pl.BlockSpec((BLK, Ho, 2 * W), lambda i: (i, 0, 0))],
        out_specs=pl.BlockSpec((BLK, Ho, Wo), lambda i: (i, 0, 0)),
        out_shape=jax.ShapeDtypeStruct((n, Ho, Wo), x.dtype),
        compiler_params=pltpu.CompilerParams(dimension_semantics=("parallel",)),
    )(xv)
    return out.reshape(B, C, Ho, Wo)
